# Initial kernel scaffold; baseline (speedup 1.0000x reference)
import jax
import jax.numpy as jnp
from jax.experimental import pallas as pl


def kernel(x, y, edge_index_no_diag, edge_attr_no_diag, ybus, edge_index, edge_attr):
    raise NotImplementedError("write your pallas kernel here")


# speedup vs baseline: 122.5827x; 122.5827x over previous
import functools

import jax
import jax.numpy as jnp
from jax import lax
from jax.experimental import pallas as pl
from jax.experimental.pallas import tpu as pltpu
from jax.experimental.pallas import tpu_sc as plsc

B = 50000
NODES = 14
N = B * NODES
E_ND = 2000000
E = 2700000

NCORES = 2
NSUB = 16
NW = NCORES * NSUB

BP = 50048
NP = BP * NODES

K0_GPT = BP // NW
K0_SG = 92
K0_NCH = K0_GPT // K0_SG
K0_SGN = K0_SG * NODES

ST_GPT = BP // NSUB
ST_SG = 184
ST_SGN = ST_SG * NODES
ST_NSC = 17
ST_TN = ST_GPT * NODES

CHUNK_ROWS = 16
ROW = 128
CHUNK_E = CHUNK_ROWS * ROW
A_NCH = 62
B_NCH = 83
EA = NSUB * A_NCH * CHUNK_E
EB = NSUB * B_NCH * CHUNK_E
A_TROWS = A_NCH * CHUNK_ROWS
B_TROWS = B_NCH * CHUNK_ROWS

_mesh = plsc.VectorSubcoreMesh(core_axis_name="c", subcore_axis_name="s")

_f32 = jnp.float32
_i32 = jnp.int32


def _iota16():
  return lax.iota(_i32, 16)


YB_BUF = K0_SG * 196 + 112
XB_BUF = 2 * K0_SGN + 16
VB0 = 1344
NB0 = 1296


@functools.partial(
    pl.kernel,
    out_type=(
        jax.ShapeDtypeStruct((NP,), _f32),
        jax.ShapeDtypeStruct((NP,), _f32),
        jax.ShapeDtypeStruct((NP,), _f32),
    ),
    mesh=_mesh,
    compiler_params=pltpu.CompilerParams(needs_layout_passes=False),
    scratch_types=[
        pltpu.VMEM((YB_BUF,), _f32),
        pltpu.VMEM((XB_BUF,), _f32),
        pltpu.VMEM((NB0,), _f32),
        pltpu.VMEM((NB0,), _f32),
        pltpu.VMEM((VB0,), _f32),
        pltpu.VMEM((NB0,), _f32),
        pltpu.VMEM((96,), _f32),
    ],
)
def _k0(x_hbm, yb_hbm, inp_hbm, rd_hbm, th0_hbm, yb_v, x_v, inp_v, rd_v,
        v_v, out_v, r0_v):
  w = lax.axis_index("s") * NCORES + lax.axis_index("c")
  lanes = _iota16()
  nsteps = NB0 // 16

  for ci in range(K0_NCH):
    g0 = w * K0_GPT + ci * K0_SG
    n0 = g0 * NODES
    pltpu.sync_copy(yb_hbm.at[pl.ds(g0 * 196, K0_SG * 196)],
                    yb_v.at[pl.ds(0, K0_SG * 196)])
    pltpu.sync_copy(x_hbm.at[pl.ds(n0 * 2, K0_SGN * 2)],
                    x_v.at[pl.ds(0, K0_SGN * 2)])

    def body1(i, _):
      d = i * 16 + lanes
      g = lax.div(d, jnp.int32(NODES))
      r = d - g * NODES
      idx_y = g * 196 + r * 15
      yb = plsc.load_gather(yb_v, [idx_y])
      rd = 1.0 / (yb * 100.0)
      x0 = plsc.load_gather(x_v, [d * 2])
      x1 = plsc.load_gather(x_v, [d * 2 + 1])
      inp = x0 - x1
      sl = pl.ds(i * 16, 16)
      inp_v[sl] = inp
      rd_v[sl] = rd
      v_v[sl] = inp * rd
      return 0

    lax.fori_loop(0, nsteps, body1, 0)

    def body2(i, _):
      g = i * 16 + lanes
      r0_v[pl.ds(i * 16, 16)] = plsc.load_gather(v_v, [g * NODES])
      return 0

    lax.fori_loop(0, 6, body2, 0)

    def body3(i, _):
      d = i * 16 + lanes
      g = lax.div(d, jnp.int32(NODES))
      sl = pl.ds(i * 16, 16)
      vv = v_v[sl]
      out_v[sl] = vv - plsc.load_gather(r0_v, [g])
      return 0

    lax.fori_loop(0, nsteps, body3, 0)

    pltpu.sync_copy(inp_v.at[pl.ds(0, K0_SGN)], inp_hbm.at[pl.ds(n0, K0_SGN)])
    pltpu.sync_copy(rd_v.at[pl.ds(0, K0_SGN)], rd_hbm.at[pl.ds(n0, K0_SGN)])
    pltpu.sync_copy(out_v.at[pl.ds(0, K0_SGN)], th0_hbm.at[pl.ds(n0, K0_SGN)])


VBUF = ST_SGN


@functools.partial(
    pl.kernel,
    out_type=(
        jax.ShapeDtypeStruct((NP,), _f32),
        jax.ShapeDtypeStruct((16, 16), _f32),
    ),
    mesh=_mesh,
    compiler_params=pltpu.CompilerParams(needs_layout_passes=False),
    scratch_types=[
        pltpu.VMEM_SHARED((NP,), _f32),
        pltpu.VMEM_SHARED((NP,), _f32),
        pltpu.VMEM((CHUNK_ROWS, ROW), _i32),
        pltpu.VMEM((CHUNK_ROWS, ROW), _i32),
        pltpu.VMEM((CHUNK_ROWS, ROW), _f32),
        pltpu.VMEM((CHUNK_ROWS, ROW), _f32),
        pltpu.VMEM((VBUF,), _f32),
        pltpu.VMEM((VBUF,), _f32),
        pltpu.VMEM((VBUF,), _f32),
        pltpu.VMEM((192,), _f32),
        pltpu.VMEM((16,), _f32),
        pltpu.SemaphoreType.DMA,
        pltpu.SemaphoreType.DMA,
    ],
)
def _step(th_hbm, inp_hbm, rd_hbm, srcA, dstA, wA, srcB, dstB, wB,
          tho_hbm, err_hbm, th_sp, acc_sp, src_v, dst_v, w_v, m_v,
          v_v, inp_v, rd_v, r0_v, err_v, gsem, ssem):
  core = lax.axis_index("c")
  s = lax.axis_index("s")
  lanes = _iota16()

  zero16 = jnp.zeros((16,), _f32)

  def zbody(i, _):
    v_v[pl.ds(i * 16, 16)] = zero16
    return 0

  lax.fori_loop(0, VBUF // 16, zbody, 0)
  for j in range(ST_NSC):
    base = s * ST_TN + j * ST_SGN
    pltpu.sync_copy(th_hbm.at[pl.ds(base, ST_SGN)], inp_v.at[pl.ds(0, ST_SGN)])
    pltpu.sync_copy(inp_v.at[pl.ds(0, ST_SGN)], th_sp.at[pl.ds(base, ST_SGN)])
    pltpu.sync_copy(v_v.at[pl.ds(0, ST_SGN)], acc_sp.at[pl.ds(base, ST_SGN)])

  plsc.subcore_barrier()

  def edge_phase(src_h, dst_h, w_h, nch, trows):
    def chunk(ci, _):
      r0 = s * trows + ci * CHUNK_ROWS
      pltpu.sync_copy(src_h.at[pl.ds(r0, CHUNK_ROWS)], src_v)
      pltpu.sync_copy(dst_h.at[pl.ds(r0, CHUNK_ROWS)], dst_v)
      pltpu.sync_copy(w_h.at[pl.ds(r0, CHUNK_ROWS)], w_v)
      gds = [
          pltpu.async_copy(th_sp.at[src_v.at[j]], m_v.at[j], gsem)
          for j in range(CHUNK_ROWS)
      ]
      for d in gds:
        d.wait()
      for j in range(CHUNK_ROWS):
        for l in range(ROW // 16):
          sl = pl.ds(l * 16, 16)
          m_v[j, sl] = m_v[j, sl] * w_v[j, sl] * 100.0
      sds = [
          pltpu.async_copy(m_v.at[j], acc_sp.at[dst_v.at[j]], ssem, add=True)
          for j in range(CHUNK_ROWS)
      ]
      for d in sds:
        d.wait()
      return 0

    lax.fori_loop(0, nch, chunk, 0)

  @pl.when(core == 0)
  def _():
    edge_phase(srcA, dstA, wA, A_NCH, A_TROWS)

  @pl.when(core == 1)
  def _():
    edge_phase(srcB, dstB, wB, B_NCH, B_TROWS)

  plsc.subcore_barrier()

  nsteps = ST_SGN // 16

  @pl.when(core == 0)
  def _():
    for j in range(ST_NSC):
      base = s * ST_TN + j * ST_SGN
      pltpu.sync_copy(acc_sp.at[pl.ds(base, ST_SGN)], v_v.at[pl.ds(0, ST_SGN)])
      pltpu.sync_copy(inp_hbm.at[pl.ds(base, ST_SGN)],
                      inp_v.at[pl.ds(0, ST_SGN)])
      pltpu.sync_copy(rd_hbm.at[pl.ds(base, ST_SGN)],
                      rd_v.at[pl.ds(0, ST_SGN)])

      def vbody(i, _):
        sl = pl.ds(i * 16, 16)
        v_v[sl] = (inp_v[sl] - v_v[sl]) * rd_v[sl]
        return 0

      lax.fori_loop(0, nsteps, vbody, 0)

      def rbody(i, _):
        g = i * 16 + lanes
        idx = jnp.where(g < ST_SG, g * NODES, 0)
        r0_v[pl.ds(i * 16, 16)] = plsc.load_gather(v_v, [idx])
        return 0

      lax.fori_loop(0, 192 // 16, rbody, 0)

      def obody(i, _):
        d = i * 16 + lanes
        g = lax.div(d, jnp.int32(NODES))
        sl = pl.ds(i * 16, 16)
        v_v[sl] = v_v[sl] - plsc.load_gather(r0_v, [g])
        return 0

      lax.fori_loop(0, nsteps, obody, 0)

      pltpu.sync_copy(v_v.at[pl.ds(0, ST_SGN)], tho_hbm.at[pl.ds(base, ST_SGN)])

  @pl.when(core == 1)
  def _():
    acc0 = jnp.zeros((16,), _f32)
    accs = []
    for j in range(ST_NSC):
      base = s * ST_TN + j * ST_SGN
      pltpu.sync_copy(acc_sp.at[pl.ds(base, ST_SGN)], v_v.at[pl.ds(0, ST_SGN)])
      pltpu.sync_copy(inp_hbm.at[pl.ds(base, ST_SGN)],
                      inp_v.at[pl.ds(0, ST_SGN)])

      def ebody(i, acc):
        sl = pl.ds(i * 16, 16)
        d = base + i * 16 + lanes
        val = jnp.abs(inp_v[sl] - v_v[sl])
        return acc + jnp.where(d < N, val, 0.0)

      acc0 = lax.fori_loop(0, nsteps, ebody, acc0)
      accs.append(acc0)
    err_v[...] = accs[-1]
    pltpu.sync_copy(err_v, err_hbm.at[s])


def _pad_edges(idx_src, idx_dst, attr, etot):
  e = idx_src.shape[0]
  pad = etot - e
  spread = (jnp.arange(pad, dtype=_i32) * 353) % N
  src = jnp.concatenate([idx_src, spread]).reshape(-1, ROW)
  dst = jnp.concatenate([idx_dst, spread]).reshape(-1, ROW)
  w = jnp.concatenate([attr, jnp.zeros((pad,), _f32)]).reshape(-1, ROW)
  return src, dst, w


def kernel(x, y, edge_index_no_diag, edge_attr_no_diag, ybus, edge_index,
           edge_attr):
  del y
  xf = jnp.pad(x.reshape(-1), (0, (NP - N) * 2))
  ybf = jnp.pad(ybus.reshape(-1), (0, (BP - B) * 196))
  inp, rd, th = _k0(xf, ybf)

  srcA, dstA, wA = _pad_edges(edge_index_no_diag[0], edge_index_no_diag[1],
                              edge_attr_no_diag, EA)
  srcB, dstB, wB = _pad_edges(edge_index[0], edge_index[1], edge_attr, EB)

  _BISECT = 0
  errs = []
  for i in range(11 if not _BISECT else 0):
    th_new, ep = _step(th, inp, rd, srcA, dstA, wA, srcB, dstB, wB)
    errs.append(ep.sum())
    if i < 10:
      th = th_new
  if _BISECT:
    errs = [th.sum() + inp.sum() + rd.sum() + wA.sum() + wB.sum()
            + srcA.sum() + srcB.sum() + dstA.sum() + dstB.sum()] * 11
  return (th[:N].reshape(-1, 1), jnp.stack(errs))


# speedup vs baseline: 193.2022x; 1.5761x over previous
import functools

import jax
import jax.numpy as jnp
from jax import lax
from jax.experimental import pallas as pl
from jax.experimental.pallas import tpu as pltpu
from jax.experimental.pallas import tpu_sc as plsc

B = 50000
NODES = 14
N = B * NODES
E_ND = 2000000
E = 2700000

NCORES = 2
NSUB = 16
NW = NCORES * NSUB

BP = 50048
NP = BP * NODES

K0_GPT = BP // NW
K0_SG = 92
K0_NCH = K0_GPT // K0_SG
K0_SGN = K0_SG * NODES

ST_GPT = BP // NSUB
ST_SG = 184
ST_SGN = ST_SG * NODES
ST_NSC = 17
ST_TN = ST_GPT * NODES

CHUNK_ROWS = 16
ROW = 128
CHUNK_E = CHUNK_ROWS * ROW
A_NCH = 62
B_NCH = 83
EA = NSUB * A_NCH * CHUNK_E
EB = NSUB * B_NCH * CHUNK_E
A_TROWS = A_NCH * CHUNK_ROWS
B_TROWS = B_NCH * CHUNK_ROWS

_mesh = plsc.VectorSubcoreMesh(core_axis_name="c", subcore_axis_name="s")

_f32 = jnp.float32
_i32 = jnp.int32


def _iota16():
  return lax.iota(_i32, 16)


YB_BUF = K0_SG * 196 + 112
XB_BUF = 2 * K0_SGN + 16
VB0 = 1344
NB0 = 1296


@functools.partial(
    pl.kernel,
    out_type=(
        jax.ShapeDtypeStruct((NP,), _f32),
        jax.ShapeDtypeStruct((NP,), _f32),
        jax.ShapeDtypeStruct((NP,), _f32),
    ),
    mesh=_mesh,
    compiler_params=pltpu.CompilerParams(needs_layout_passes=False),
    scratch_types=[
        pltpu.VMEM((YB_BUF,), _f32),
        pltpu.VMEM((XB_BUF,), _f32),
        pltpu.VMEM((NB0,), _f32),
        pltpu.VMEM((NB0,), _f32),
        pltpu.VMEM((VB0,), _f32),
        pltpu.VMEM((NB0,), _f32),
        pltpu.VMEM((96,), _f32),
    ],
)
def _k0(x_hbm, yb_hbm, inp_hbm, rd_hbm, th0_hbm, yb_v, x_v, inp_v, rd_v,
        v_v, out_v, r0_v):
  w = lax.axis_index("s") * NCORES + lax.axis_index("c")
  lanes = _iota16()
  nsteps = NB0 // 16

  for ci in range(K0_NCH):
    g0 = w * K0_GPT + ci * K0_SG
    n0 = g0 * NODES
    pltpu.sync_copy(yb_hbm.at[pl.ds(g0 * 196, K0_SG * 196)],
                    yb_v.at[pl.ds(0, K0_SG * 196)])
    pltpu.sync_copy(x_hbm.at[pl.ds(n0 * 2, K0_SGN * 2)],
                    x_v.at[pl.ds(0, K0_SGN * 2)])

    def body1(i, _):
      d = i * 16 + lanes
      g = lax.div(d, jnp.int32(NODES))
      r = d - g * NODES
      idx_y = g * 196 + r * 15
      yb = plsc.load_gather(yb_v, [idx_y])
      rd = 1.0 / (yb * 100.0)
      x0 = plsc.load_gather(x_v, [d * 2])
      x1 = plsc.load_gather(x_v, [d * 2 + 1])
      inp = x0 - x1
      sl = pl.ds(i * 16, 16)
      inp_v[sl] = inp
      rd_v[sl] = rd
      v_v[sl] = inp * rd
      return 0

    lax.fori_loop(0, nsteps, body1, 0)

    def body2(i, _):
      g = i * 16 + lanes
      r0_v[pl.ds(i * 16, 16)] = plsc.load_gather(v_v, [g * NODES])
      return 0

    lax.fori_loop(0, 6, body2, 0)

    def body3(i, _):
      d = i * 16 + lanes
      g = lax.div(d, jnp.int32(NODES))
      sl = pl.ds(i * 16, 16)
      vv = v_v[sl]
      out_v[sl] = vv - plsc.load_gather(r0_v, [g])
      return 0

    lax.fori_loop(0, nsteps, body3, 0)

    pltpu.sync_copy(inp_v.at[pl.ds(0, K0_SGN)], inp_hbm.at[pl.ds(n0, K0_SGN)])
    pltpu.sync_copy(rd_v.at[pl.ds(0, K0_SGN)], rd_hbm.at[pl.ds(n0, K0_SGN)])
    pltpu.sync_copy(out_v.at[pl.ds(0, K0_SGN)], th0_hbm.at[pl.ds(n0, K0_SGN)])


VBUF = ST_SGN


@functools.partial(
    pl.kernel,
    out_type=(
        jax.ShapeDtypeStruct((NP,), _f32),
        jax.ShapeDtypeStruct((16, 16), _f32),
    ),
    mesh=_mesh,
    compiler_params=pltpu.CompilerParams(needs_layout_passes=False),
    scratch_types=[
        pltpu.VMEM_SHARED((NP,), _f32),
        pltpu.VMEM_SHARED((NP,), _f32),
        pltpu.VMEM((CHUNK_E,), _i32),
        pltpu.VMEM((CHUNK_E,), _i32),
        pltpu.VMEM((CHUNK_E,), _f32),
        pltpu.VMEM((CHUNK_E,), _f32),
        pltpu.VMEM((CHUNK_E,), _i32),
        pltpu.VMEM((CHUNK_E,), _i32),
        pltpu.VMEM((CHUNK_E,), _f32),
        pltpu.VMEM((CHUNK_E,), _f32),
        pltpu.VMEM((4096,), _f32),
        pltpu.VMEM((4096,), _f32),
        pltpu.VMEM((VBUF,), _f32),
        pltpu.VMEM((VBUF,), _f32),
        pltpu.VMEM((VBUF,), _f32),
        pltpu.VMEM((192,), _f32),
        pltpu.VMEM((16,), _f32),
        pltpu.SemaphoreType.DMA,
        pltpu.SemaphoreType.DMA,
        pltpu.SemaphoreType.DMA,
    ],
)
def _step(th_hbm, inp_hbm, rd_hbm, srcA, dstA, wA, srcB, dstB, wB,
          tho_hbm, err_hbm, th_sp, acc_sp, src_v0, dst_v0, w_v0, m_v0,
          src_v1, dst_v1, w_v1, m_v1, b_v, b2_v,
          v_v, inp_v, rd_v, r0_v, err_v, gsem, ssem, isem):
  core = lax.axis_index("c")
  s = lax.axis_index("s")
  lanes = _iota16()

  zero16 = jnp.zeros((16,), _f32)

  def zbody(i, _):
    b2_v[pl.ds(i * 16, 16)] = zero16
    return 0

  lax.fori_loop(0, 4096 // 16, zbody, 0)
  t0 = s * ST_TN
  NPC = 11
  rem = ST_TN - 10 * 4096
  szs = (4096,) * 10 + (rem,)
  zds = [
      pltpu.async_copy(b2_v.at[pl.ds(0, szs[j])],
                       acc_sp.at[pl.ds(t0 + j * 4096, szs[j])], isem)
      for j in range(NPC)
  ]
  for d in zds:
    d.wait()
  BB = (b_v, b2_v)
  d_in = pltpu.async_copy(th_hbm.at[pl.ds(t0, 4096)], b_v, gsem)
  d_out = None
  for j in range(NPC):
    nb = BB[j % 2]
    d_in.wait()
    if j >= 1:
      pltpu.make_async_copy(b_v.at[pl.ds(0, szs[j - 1])],
                            th_sp.at[pl.ds(t0 + (j - 1) * 4096, szs[j - 1])],
                            ssem).wait()
    if j < NPC - 1:
      d_in = pltpu.async_copy(
          th_hbm.at[pl.ds(t0 + (j + 1) * 4096, szs[j + 1])],
          BB[(j + 1) % 2].at[pl.ds(0, szs[j + 1])], gsem)
    d_out = pltpu.async_copy(nb.at[pl.ds(0, szs[j])],
                             th_sp.at[pl.ds(t0 + j * 4096, szs[j])], ssem)
  d_out.wait()

  plsc.subcore_barrier()

  SV = (src_v0, src_v1)
  DV = (dst_v0, dst_v1)
  WV = (w_v0, w_v1)
  MV = (m_v0, m_v1)

  def edge_phase(src_h, dst_h, w_h, nch, ept):
    def inputs(c, b):
      e0 = s * ept + c * CHUNK_E
      pltpu.async_copy(src_h.at[pl.ds(e0, CHUNK_E)], SV[b], isem)
      pltpu.async_copy(dst_h.at[pl.ds(e0, CHUNK_E)], DV[b], isem)
      pltpu.async_copy(w_h.at[pl.ds(e0, CHUNK_E)], WV[b], isem)

    def wait_inputs(b):
      for h, v in ((src_h, SV[b]), (dst_h, DV[b]), (w_h, WV[b])):
        pltpu.make_async_copy(h.at[pl.ds(0, CHUNK_E)], v, isem).wait()

    def gather(b):
      pltpu.async_copy(th_sp.at[SV[b]], MV[b], gsem)

    def wait_gather(b):
      pltpu.make_async_copy(th_sp.at[SV[b]], MV[b], gsem).wait()

    def mul(b):
      for l in range(CHUNK_E // 16):
        sl = pl.ds(l * 16, 16)
        MV[b][sl] = MV[b][sl] * WV[b][sl] * 100.0

    def scatter(b):
      pltpu.async_copy(MV[b], acc_sp.at[DV[b]], ssem, add=True)

    def wait_scatter(b):
      pltpu.make_async_copy(MV[b], acc_sp.at[DV[b]], ssem).wait()

    inputs(0, 0)

    def body(k, _):
      c0 = 2 * k
      wait_inputs(0)
      gather(0)

      @pl.when(k > 0)
      def _():
        wait_scatter(1)

      inputs(c0 + 1, 1)
      wait_gather(0)
      mul(0)
      scatter(0)
      wait_inputs(1)
      gather(1)
      wait_scatter(0)
      inputs(c0 + 2, 0)
      wait_gather(1)
      mul(1)
      scatter(1)
      return 0

    lax.fori_loop(0, nch // 2, body, 0)
    if nch % 2 == 1:
      wait_inputs(0)
      gather(0)
      wait_scatter(1)
      wait_gather(0)
      mul(0)
      scatter(0)
      wait_scatter(0)
    else:
      wait_scatter(1)
      wait_inputs(0)

  @pl.when(core == 0)
  def _():
    edge_phase(srcA, dstA, wA, A_NCH, A_NCH * CHUNK_E)

  @pl.when(core == 1)
  def _():
    edge_phase(srcB, dstB, wB, B_NCH, B_NCH * CHUNK_E)

  plsc.subcore_barrier()

  nsteps = ST_SGN // 16

  @pl.when(core == 0)
  def _():
    for j in range(ST_NSC):
      base = s * ST_TN + j * ST_SGN
      pltpu.sync_copy(acc_sp.at[pl.ds(base, ST_SGN)], v_v.at[pl.ds(0, ST_SGN)])
      pltpu.sync_copy(inp_hbm.at[pl.ds(base, ST_SGN)],
                      inp_v.at[pl.ds(0, ST_SGN)])
      pltpu.sync_copy(rd_hbm.at[pl.ds(base, ST_SGN)],
                      rd_v.at[pl.ds(0, ST_SGN)])

      def vbody(i, _):
        sl = pl.ds(i * 16, 16)
        v_v[sl] = (inp_v[sl] - v_v[sl]) * rd_v[sl]
        return 0

      lax.fori_loop(0, nsteps, vbody, 0)

      def rbody(i, _):
        g = i * 16 + lanes
        idx = jnp.where(g < ST_SG, g * NODES, 0)
        r0_v[pl.ds(i * 16, 16)] = plsc.load_gather(v_v, [idx])
        return 0

      lax.fori_loop(0, 192 // 16, rbody, 0)

      def obody(i, _):
        d = i * 16 + lanes
        g = lax.div(d, jnp.int32(NODES))
        sl = pl.ds(i * 16, 16)
        v_v[sl] = v_v[sl] - plsc.load_gather(r0_v, [g])
        return 0

      lax.fori_loop(0, nsteps, obody, 0)

      pltpu.sync_copy(v_v.at[pl.ds(0, ST_SGN)], tho_hbm.at[pl.ds(base, ST_SGN)])

  @pl.when(core == 1)
  def _():
    acc0 = jnp.zeros((16,), _f32)
    accs = []
    for j in range(ST_NSC):
      base = s * ST_TN + j * ST_SGN
      pltpu.sync_copy(acc_sp.at[pl.ds(base, ST_SGN)], v_v.at[pl.ds(0, ST_SGN)])
      pltpu.sync_copy(inp_hbm.at[pl.ds(base, ST_SGN)],
                      inp_v.at[pl.ds(0, ST_SGN)])

      def ebody(i, acc):
        sl = pl.ds(i * 16, 16)
        d = base + i * 16 + lanes
        val = jnp.abs(inp_v[sl] - v_v[sl])
        return acc + jnp.where(d < N, val, 0.0)

      acc0 = lax.fori_loop(0, nsteps, ebody, acc0)
      accs.append(acc0)
    err_v[...] = accs[-1]
    pltpu.sync_copy(err_v, err_hbm.at[s])


def _pad_edges(idx_src, idx_dst, attr, etot):
  e = idx_src.shape[0]
  pad = etot - e
  spread = (jnp.arange(pad, dtype=_i32) * 353) % N
  src = jnp.concatenate([idx_src, spread])
  dst = jnp.concatenate([idx_dst, spread])
  w = jnp.concatenate([attr, jnp.zeros((pad,), _f32)])
  return src, dst, w


def kernel(x, y, edge_index_no_diag, edge_attr_no_diag, ybus, edge_index,
           edge_attr):
  del y
  xf = jnp.pad(x.reshape(-1), (0, (NP - N) * 2))
  ybf = jnp.pad(ybus.reshape(-1), (0, (BP - B) * 196))
  inp, rd, th = _k0(xf, ybf)

  srcA, dstA, wA = _pad_edges(edge_index_no_diag[0], edge_index_no_diag[1],
                              edge_attr_no_diag, EA + CHUNK_E)
  srcB, dstB, wB = _pad_edges(edge_index[0], edge_index[1], edge_attr,
                              EB + CHUNK_E)

  _BISECT = 0
  errs = []
  for i in range(11 if not _BISECT else 0):
    th_new, ep = _step(th, inp, rd, srcA, dstA, wA, srcB, dstB, wB)
    errs.append(ep.sum())
    if i < 10:
      th = th_new
  if _BISECT:
    errs = [th.sum() + inp.sum() + rd.sum() + wA.sum() + wB.sum()
            + srcA.sum() + srcB.sum() + dstA.sum() + dstB.sum()] * 11
  return (th[:N].reshape(-1, 1), jnp.stack(errs))


# speedup vs baseline: 200.7344x; 1.0390x over previous
import functools

import jax
import jax.numpy as jnp
from jax import lax
from jax.experimental import pallas as pl
from jax.experimental.pallas import tpu as pltpu
from jax.experimental.pallas import tpu_sc as plsc

B = 50000
NODES = 14
N = B * NODES
E_ND = 2000000
E = 2700000

NCORES = 2
NSUB = 16
NW = NCORES * NSUB

BP = 50048
NP = BP * NODES

K0_GPT = BP // NW
K0_SG = 92
K0_NCH = K0_GPT // K0_SG
K0_SGN = K0_SG * NODES

ST_GPT = BP // NSUB
ST_SG = 184
ST_SGN = ST_SG * NODES
ST_NSC = 17
ST_TN = ST_GPT * NODES

CHUNK_ROWS = 16
ROW = 128
CHUNK_E = CHUNK_ROWS * ROW
A_NCH = 62
B_NCH = 83
EA = NSUB * A_NCH * CHUNK_E
EB = NSUB * B_NCH * CHUNK_E
A_TROWS = A_NCH * CHUNK_ROWS
B_TROWS = B_NCH * CHUNK_ROWS

_mesh = plsc.VectorSubcoreMesh(core_axis_name="c", subcore_axis_name="s")

_f32 = jnp.float32
_i32 = jnp.int32


def _iota16():
  return lax.iota(_i32, 16)


YB_BUF = K0_SG * 196 + 112
XB_BUF = 2 * K0_SGN + 16
VB0 = 1344
NB0 = 1296


@functools.partial(
    pl.kernel,
    out_type=(
        jax.ShapeDtypeStruct((NP,), _f32),
        jax.ShapeDtypeStruct((NP,), _f32),
        jax.ShapeDtypeStruct((NP,), _f32),
    ),
    mesh=_mesh,
    compiler_params=pltpu.CompilerParams(needs_layout_passes=False),
    scratch_types=[
        pltpu.VMEM((YB_BUF,), _f32),
        pltpu.VMEM((XB_BUF,), _f32),
        pltpu.VMEM((NB0,), _f32),
        pltpu.VMEM((NB0,), _f32),
        pltpu.VMEM((VB0,), _f32),
        pltpu.VMEM((NB0,), _f32),
        pltpu.VMEM((96,), _f32),
    ],
)
def _k0(x_hbm, yb_hbm, inp_hbm, rd_hbm, th0_hbm, yb_v, x_v, inp_v, rd_v,
        v_v, out_v, r0_v):
  w = lax.axis_index("s") * NCORES + lax.axis_index("c")
  lanes = _iota16()
  nsteps = NB0 // 16

  for ci in range(K0_NCH):
    g0 = w * K0_GPT + ci * K0_SG
    n0 = g0 * NODES
    pltpu.sync_copy(yb_hbm.at[pl.ds(g0 * 196, K0_SG * 196)],
                    yb_v.at[pl.ds(0, K0_SG * 196)])
    pltpu.sync_copy(x_hbm.at[pl.ds(n0 * 2, K0_SGN * 2)],
                    x_v.at[pl.ds(0, K0_SGN * 2)])

    def body1(i, _):
      d = i * 16 + lanes
      g = lax.div(d, jnp.int32(NODES))
      r = d - g * NODES
      idx_y = g * 196 + r * 15
      yb = plsc.load_gather(yb_v, [idx_y])
      rd = 1.0 / (yb * 100.0)
      x0 = plsc.load_gather(x_v, [d * 2])
      x1 = plsc.load_gather(x_v, [d * 2 + 1])
      inp = x0 - x1
      sl = pl.ds(i * 16, 16)
      inp_v[sl] = inp
      rd_v[sl] = rd
      v_v[sl] = inp * rd
      return 0

    lax.fori_loop(0, nsteps, body1, 0)

    def body2(i, _):
      g = i * 16 + lanes
      r0_v[pl.ds(i * 16, 16)] = plsc.load_gather(v_v, [g * NODES])
      return 0

    lax.fori_loop(0, 6, body2, 0)

    def body3(i, _):
      d = i * 16 + lanes
      g = lax.div(d, jnp.int32(NODES))
      sl = pl.ds(i * 16, 16)
      vv = v_v[sl]
      out_v[sl] = vv - plsc.load_gather(r0_v, [g])
      return 0

    lax.fori_loop(0, nsteps, body3, 0)

    pltpu.sync_copy(inp_v.at[pl.ds(0, K0_SGN)], inp_hbm.at[pl.ds(n0, K0_SGN)])
    pltpu.sync_copy(rd_v.at[pl.ds(0, K0_SGN)], rd_hbm.at[pl.ds(n0, K0_SGN)])
    pltpu.sync_copy(out_v.at[pl.ds(0, K0_SGN)], th0_hbm.at[pl.ds(n0, K0_SGN)])


VBUF = ST_SGN


@functools.partial(
    pl.kernel,
    out_type=(
        jax.ShapeDtypeStruct((NP,), _f32),
        jax.ShapeDtypeStruct((176, 16), _f32),
    ),
    mesh=_mesh,
    compiler_params=pltpu.CompilerParams(needs_layout_passes=False),
    scratch_types=[
        pltpu.VMEM_SHARED((NP,), _f32),
        pltpu.VMEM_SHARED((NP,), _f32),
        pltpu.VMEM((CHUNK_E,), _i32),
        pltpu.VMEM((CHUNK_E,), _i32),
        pltpu.VMEM((CHUNK_E,), _f32),
        pltpu.VMEM((CHUNK_E,), _f32),
        pltpu.VMEM((CHUNK_E,), _i32),
        pltpu.VMEM((CHUNK_E,), _i32),
        pltpu.VMEM((CHUNK_E,), _f32),
        pltpu.VMEM((CHUNK_E,), _f32),
        pltpu.VMEM((4096,), _f32),
        pltpu.VMEM((4096,), _f32),
        pltpu.VMEM((VBUF,), _f32),
        pltpu.VMEM((VBUF,), _f32),
        pltpu.VMEM((VBUF,), _f32),
        pltpu.VMEM((192,), _f32),
        pltpu.VMEM((16,), _f32),
        pltpu.SemaphoreType.DMA,
        pltpu.SemaphoreType.DMA,
        pltpu.SemaphoreType.DMA,
    ],
)
def _step(th_hbm, inp_hbm, rd_hbm, srcA, dstA, wA, srcB, dstB, wB,
          tho_hbm, err_hbm, th_sp, acc_sp, src_v0, dst_v0, w_v0, m_v0,
          src_v1, dst_v1, w_v1, m_v1, b_v, b2_v,
          v_v, inp_v, rd_v, r0_v, err_v, gsem, ssem, isem):
  core = lax.axis_index("c")
  s = lax.axis_index("s")
  lanes = _iota16()
  zero16 = jnp.zeros((16,), _f32)
  t0 = s * ST_TN
  NPC = 11
  rem = ST_TN - 10 * 4096
  szs = (4096,) * 10 + (rem,)

  def stage_theta(src_hbm):
    BB = (b_v, b2_v)
    d_in = pltpu.async_copy(src_hbm.at[pl.ds(t0, 4096)], b_v, gsem)
    d_out = None
    for j in range(NPC):
      nb = BB[j % 2]
      d_in.wait()
      if j >= 1:
        pltpu.make_async_copy(b_v.at[pl.ds(0, szs[j - 1])],
                              th_sp.at[pl.ds(t0 + (j - 1) * 4096, szs[j - 1])],
                              ssem).wait()
      if j < NPC - 1:
        d_in = pltpu.async_copy(
            src_hbm.at[pl.ds(t0 + (j + 1) * 4096, szs[j + 1])],
            BB[(j + 1) % 2].at[pl.ds(0, szs[j + 1])], gsem)
      d_out = pltpu.async_copy(nb.at[pl.ds(0, szs[j])],
                               th_sp.at[pl.ds(t0 + j * 4096, szs[j])], ssem)
    d_out.wait()

  def iter_body(it, _carry):
    def zbody(i, _):
      b2_v[pl.ds(i * 16, 16)] = zero16
      return 0

    lax.fori_loop(0, 4096 // 16, zbody, 0)
    zds = [
        pltpu.async_copy(b2_v.at[pl.ds(0, szs[j])],
                         acc_sp.at[pl.ds(t0 + j * 4096, szs[j])], isem)
        for j in range(NPC)
    ]
    for d in zds:
      d.wait()

    @pl.when(it == 0)
    def _():
      stage_theta(th_hbm)

    @pl.when((core == 1) & (it > 0))
    def _():
      stage_theta(tho_hbm)

    plsc.subcore_barrier()
    return it

  SV = (src_v0, src_v1)
  DV = (dst_v0, dst_v1)
  WV = (w_v0, w_v1)
  MV = (m_v0, m_v1)

  def edge_phase(src_h, dst_h, w_h, nch, ept):
    def inputs(c, b):
      e0 = s * ept + c * CHUNK_E
      pltpu.async_copy(src_h.at[pl.ds(e0, CHUNK_E)], SV[b], isem)
      pltpu.async_copy(dst_h.at[pl.ds(e0, CHUNK_E)], DV[b], isem)
      pltpu.async_copy(w_h.at[pl.ds(e0, CHUNK_E)], WV[b], isem)

    def wait_inputs(b):
      for h, v in ((src_h, SV[b]), (dst_h, DV[b]), (w_h, WV[b])):
        pltpu.make_async_copy(h.at[pl.ds(0, CHUNK_E)], v, isem).wait()

    def gather(b):
      pltpu.async_copy(th_sp.at[SV[b]], MV[b], gsem)

    def wait_gather(b):
      pltpu.make_async_copy(th_sp.at[SV[b]], MV[b], gsem).wait()

    def mul(b):
      for l in range(CHUNK_E // 16):
        sl = pl.ds(l * 16, 16)
        MV[b][sl] = MV[b][sl] * WV[b][sl] * 100.0

    def scatter(b):
      pltpu.async_copy(MV[b], acc_sp.at[DV[b]], ssem, add=True)

    def wait_scatter(b):
      pltpu.make_async_copy(MV[b], acc_sp.at[DV[b]], ssem).wait()

    inputs(0, 0)

    def body(k, _):
      c0 = 2 * k
      wait_inputs(0)
      gather(0)

      @pl.when(k > 0)
      def _():
        wait_scatter(1)

      inputs(c0 + 1, 1)
      wait_gather(0)
      mul(0)
      scatter(0)
      wait_inputs(1)
      gather(1)
      wait_scatter(0)
      inputs(c0 + 2, 0)
      wait_gather(1)
      mul(1)
      scatter(1)
      return 0

    lax.fori_loop(0, nch // 2, body, 0)
    if nch % 2 == 1:
      wait_inputs(0)
      gather(0)
      wait_scatter(1)
      wait_gather(0)
      mul(0)
      scatter(0)
      wait_scatter(0)
    else:
      wait_scatter(1)
      wait_inputs(0)

  def iter_edges(it):
    @pl.when((core == 0) & (it < 10))
    def _():
      edge_phase(srcA, dstA, wA, A_NCH, A_NCH * CHUNK_E)

    @pl.when(core == 1)
    def _():
      edge_phase(srcB, dstB, wB, B_NCH, B_NCH * CHUNK_E)

    plsc.subcore_barrier()

  nsteps = ST_SGN // 16

  def iter_readback(it):
    @pl.when((core == 0) & (it < 10))
    def _():
      for j in range(ST_NSC):
        base = s * ST_TN + j * ST_SGN
        pltpu.sync_copy(acc_sp.at[pl.ds(base, ST_SGN)],
                        v_v.at[pl.ds(0, ST_SGN)])
        pltpu.sync_copy(inp_hbm.at[pl.ds(base, ST_SGN)],
                        inp_v.at[pl.ds(0, ST_SGN)])
        pltpu.sync_copy(rd_hbm.at[pl.ds(base, ST_SGN)],
                        rd_v.at[pl.ds(0, ST_SGN)])

        def vbody(i, _):
          sl = pl.ds(i * 16, 16)
          v_v[sl] = (inp_v[sl] - v_v[sl]) * rd_v[sl]
          return 0

        lax.fori_loop(0, nsteps, vbody, 0)

        def rbody(i, _):
          g = i * 16 + lanes
          idx = jnp.where(g < ST_SG, g * NODES, 0)
          r0_v[pl.ds(i * 16, 16)] = plsc.load_gather(v_v, [idx])
          return 0

        lax.fori_loop(0, 192 // 16, rbody, 0)

        def obody(i, _):
          d = i * 16 + lanes
          g = lax.div(d, jnp.int32(NODES))
          sl = pl.ds(i * 16, 16)
          v_v[sl] = v_v[sl] - plsc.load_gather(r0_v, [g])
          return 0

        lax.fori_loop(0, nsteps, obody, 0)

        pltpu.sync_copy(v_v.at[pl.ds(0, ST_SGN)],
                        tho_hbm.at[pl.ds(base, ST_SGN)])
        pltpu.sync_copy(v_v.at[pl.ds(0, ST_SGN)],
                        th_sp.at[pl.ds(base, ST_SGN)])

    @pl.when(core == 1)
    def _():
      acc0 = jnp.zeros((16,), _f32)
      accs = []
      for j in range(ST_NSC):
        base = s * ST_TN + j * ST_SGN
        pltpu.sync_copy(acc_sp.at[pl.ds(base, ST_SGN)],
                        v_v.at[pl.ds(0, ST_SGN)])
        pltpu.sync_copy(inp_hbm.at[pl.ds(base, ST_SGN)],
                        inp_v.at[pl.ds(0, ST_SGN)])

        def ebody(i, acc):
          sl = pl.ds(i * 16, 16)
          d = base + i * 16 + lanes
          val = jnp.abs(inp_v[sl] - v_v[sl])
          return acc + jnp.where(d < N, val, 0.0)

        acc0 = lax.fori_loop(0, nsteps, ebody, acc0)
        accs.append(acc0)
      err_v[...] = accs[-1]
      pltpu.sync_copy(err_v, err_hbm.at[it * 16 + s])

    plsc.subcore_barrier()

  def whole_iter(it, carry):
    iter_body(it, carry)
    iter_edges(it)
    iter_readback(it)
    return carry

  lax.fori_loop(0, 11, whole_iter, 0)


def _pad_edges(idx_src, idx_dst, attr, etot):
  e = idx_src.shape[0]
  pad = etot - e
  spread = (jnp.arange(pad, dtype=_i32) * 353) % N
  src = jnp.concatenate([idx_src, spread])
  dst = jnp.concatenate([idx_dst, spread])
  w = jnp.concatenate([attr, jnp.zeros((pad,), _f32)])
  return src, dst, w


def kernel(x, y, edge_index_no_diag, edge_attr_no_diag, ybus, edge_index,
           edge_attr):
  del y
  xf = jnp.pad(x.reshape(-1), (0, (NP - N) * 2))
  ybf = jnp.pad(ybus.reshape(-1), (0, (BP - B) * 196))
  inp, rd, th = _k0(xf, ybf)

  srcA, dstA, wA = _pad_edges(edge_index_no_diag[0], edge_index_no_diag[1],
                              edge_attr_no_diag, EA + CHUNK_E)
  srcB, dstB, wB = _pad_edges(edge_index[0], edge_index[1], edge_attr,
                              EB + CHUNK_E)

  th_out, ep = _step(th, inp, rd, srcA, dstA, wA, srcB, dstB, wB)
  errs = ep.reshape(11, 256).sum(axis=1)
  return (th_out[:N].reshape(-1, 1), errs)
